# baseline (speedup 1.0000x reference)
import functools

import jax
import jax.numpy as jnp
from jax import lax
from jax.experimental import pallas as pl
from jax.experimental.pallas import tpu as pltpu
from jax.experimental.pallas import tpu_sc as plsc


def _make_gather(V, D, B):
    info = plsc.get_sparse_core_info()
    NC, NS = info.num_cores, info.num_subcores
    NW = NC * NS
    assert D == 64 and V % 8 == 0 and B % 16 == 0
    SEG = (V // NW) // 8 * 8
    R = 192
    NSLAB = -(-(V - SEG * (NW - 1)) // R)
    NPAIR = -(-NSLAB // 2)
    STG = 128
    CAP = B + 16
    NIV = B // 16
    SENTINEL = jnp.int32(2**31 - 1)
    mesh = plsc.VectorSubcoreMesh(core_axis_name="c", subcore_axis_name="s")

    @functools.partial(
        pl.kernel,
        mesh=mesh,
        out_type=jax.ShapeDtypeStruct((B + 8, 2 * D), jnp.float32),
        compiler_params=pltpu.CompilerParams(needs_layout_passes=False),
        scratch_types=[
            pltpu.VMEM((B,), jnp.int32),
            pltpu.VMEM((CAP,), jnp.int32),
            pltpu.VMEM((CAP,), jnp.int32),
            pltpu.VMEM((R, D), jnp.float32),
            pltpu.VMEM((R, D), jnp.float32),
            pltpu.VMEM((STG, 2 * D), jnp.float32),
            pltpu.VMEM((STG,), jnp.int32),
            pltpu.SemaphoreType.DMA,
            pltpu.SemaphoreType.DMA,
            pltpu.SemaphoreType.DMA,
        ],
    )
    def k(table_hbm, idx_hbm, out_hbm, idx_v, midx_v, mpos_v, slab_a, slab_b,
          stage_v, opos_v, sem_a, sem_b, sem_s):
        wid = lax.axis_index("s") * NC + lax.axis_index("c")
        lo = wid * SEG
        hi = jnp.where(wid == NW - 1, V, lo + SEG)
        dump = jnp.int32(B) + lax.rem(wid, 8)
        lane = lax.iota(jnp.int32, 16)

        pltpu.sync_copy(idx_hbm, idx_v)

        def collect(kv, off):
            ivec = idx_v[pl.ds(kv * 16, 16)]
            pvec = kv * 16 + lane
            mb = jnp.logical_and(ivec >= lo, ivec < hi)
            m = jnp.where(mb, 1, 0).astype(jnp.int32)
            cum = plsc.cumsum(m)
            tot = cum[15]

            @pl.when(tot > 0)
            def _():
                plsc.store_compressed(midx_v.at[pl.ds(off, 16)], ivec, mask=mb)
                plsc.store_compressed(mpos_v.at[pl.ds(off, 16)], pvec, mask=mb)

            return off + tot

        cnt = lax.fori_loop(0, NIV, collect, jnp.int32(0), unroll=False)
        midx_v[pl.ds(cnt, 16)] = jnp.full((16,), SENTINEL, jnp.int32)
        nvec = (cnt + 15) // 16

        for q in range(STG // 16):
            opos_v[pl.ds(q * 16, 16)] = jnp.full((16,), 1, jnp.int32) * dump

        def slab_start(s):
            return jnp.minimum(lo + s * R, V - R)

        def fire(s, buf, sem):
            return pltpu.async_copy(
                table_hbm.at[pl.ds(slab_start(s), R), :], buf, sem
            )

        def drain(buf, sem):
            pltpu.make_async_copy(
                table_hbm.at[pl.ds(0, R), :], buf, sem
            ).wait()

        def flush():
            pltpu.async_copy(stage_v, out_hbm.at[opos_v], sem_s).wait()
            for q in range(STG // 16):
                opos_v[pl.ds(q * 16, 16)] = jnp.full((16,), 1, jnp.int32) * dump

        def process(buf, r0, sc_in):
            def body(kv, sc):
                mvec = midx_v[pl.ds(kv * 16, 16)]
                pvec = mpos_v[pl.ds(kv * 16, 16)]
                mb = jnp.logical_and(mvec >= r0, mvec < r0 + R)
                m = jnp.where(mb, 1, 0).astype(jnp.int32)
                cum = plsc.cumsum(m)
                tot = cum[15]
                need_flush = jnp.logical_and(tot > 0, sc + tot > STG)

                @pl.when(need_flush)
                def _():
                    flush()

                sc0 = jnp.where(need_flush, 0, sc)

                @pl.when(tot > 0)
                def _():
                    slot_vec = sc0 + cum - 1
                    plsc.store_scatter(opos_v, [slot_vec], pvec, mask=mb)
                    for r in range(16):
                        @pl.when(m[r] > 0)
                        def _():
                            row = mvec[r] - r0
                            slot = sc0 + cum[r] - 1
                            for c in range(D // 16):
                                stage_v[slot, pl.ds(c * 16, 16)] = (
                                    buf[row, pl.ds(c * 16, 16)]
                                )

                return sc0 + tot

            return lax.fori_loop(0, nvec, body, sc_in, unroll=False)

        h_a = fire(0, slab_a, sem_a)
        h_b = fire(1, slab_b, sem_b)
        del h_a, h_b

        def pair(p, sc):
            drain(slab_a, sem_a)
            sc = process(slab_a, slab_start(2 * p), sc)
            fire(2 * p + 2, slab_a, sem_a)
            drain(slab_b, sem_b)
            sc = process(slab_b, slab_start(2 * p + 1), sc)
            fire(2 * p + 3, slab_b, sem_b)
            return sc

        sc = lax.fori_loop(0, NPAIR, pair, jnp.int32(0), unroll=False)
        drain(slab_a, sem_a)
        drain(slab_b, sem_b)

        @pl.when(sc > 0)
        def _():
            flush()

    def run(full_tensor, idx32):
        out_pad = k(full_tensor, idx32)
        return lax.slice(out_pad, (0, 0), (B, D))

    return run


def kernel(full_tensor, indices):
    V, D = full_tensor.shape
    (B,) = indices.shape
    idx32 = indices.astype(jnp.int32)
    return _make_gather(V, D, B)(full_tensor, idx32)
